# Initial kernel scaffold; baseline (speedup 1.0000x reference)
import jax
import jax.numpy as jnp
from jax.experimental import pallas as pl


def kernel(R, offsets, idx_i, idx_j):
    raise NotImplementedError("write your pallas kernel here")


# speedup vs baseline: 1.3175x; 1.3175x over previous
import jax
import jax.numpy as jnp
from jax import lax
from jax.experimental import pallas as pl
from jax.experimental.pallas import tpu as pltpu
from jax.experimental.pallas import tpu_sc as plsc

N_CORES = 2
N_SUBCORES = 16
N_WORKERS = N_CORES * N_SUBCORES

N_ATOMS_WORDS = 75000
B = 2000
GROUPS = (B * 3) // 48

HI_MASK = -65536


def _body(rw_hbm, off_hbm, idxi_hbm, idxj_hbm, out_hbm,
          rw_v, idxi_v, idxj_v, off_v, out_v):
    wid = lax.axis_index("s") * N_CORES + lax.axis_index("c")
    pairs_per_w = idxi_hbm.shape[0] // N_WORKERS
    nblocks = pairs_per_w // B
    w_base = wid * pairs_per_w

    pltpu.sync_copy(rw_hbm, rw_v)

    lanes = lax.iota(jnp.int32, 16)
    pp = []
    cc = []
    for k in range(3):
        g = lanes + (16 * k)
        p = (g * 21846) >> 16
        pp.append(p)
        cc.append(g - 3 * p)

    def _elem(table, idx_v, p_vec, c_vec):
        pv = plsc.load_gather(idx_v, [p_vec])
        e = pv * 3 + c_vec
        a = plsc.load_gather(table, [e >> 1])
        bits = jnp.where((e & 1) == 1, a & HI_MASK, a << 16)
        return plsc.bitcast(bits, jnp.float32)

    @pl.loop(0, nblocks)
    def _block(b):
        pair_base = w_base + b * B

        pltpu.sync_copy(idxi_hbm.at[pl.ds(pair_base, B)], idxi_v)
        pltpu.sync_copy(idxj_hbm.at[pl.ds(pair_base, B)], idxj_v)
        pltpu.sync_copy(off_hbm.at[pl.ds(3 * pair_base, 3 * B)], off_v)

        @pl.loop(0, GROUPS)
        def _group(g):
            prow = 16 * g
            for k in range(3):
                w = 48 * g + 16 * k
                vj = _elem(rw_v, idxj_v, prow + pp[k], cc[k])
                vi = _elem(rw_v, idxi_v, prow + pp[k], cc[k])
                out_v[pl.ds(w, 16)] = vj - vi + off_v[pl.ds(w, 16)]

        pltpu.sync_copy(out_v, out_hbm.at[pl.ds(3 * pair_base, 3 * B)])


def kernel(R, offsets, idx_i, idx_j):
    n_pairs = idx_i.shape[0]
    rw = lax.bitcast_convert_type(
        R.astype(jnp.bfloat16).reshape(N_ATOMS_WORDS, 2), jnp.int32)
    off_flat = offsets.reshape(-1)
    idx_i = idx_i.astype(jnp.int32)
    idx_j = idx_j.astype(jnp.int32)

    mesh = plsc.VectorSubcoreMesh(core_axis_name="c", subcore_axis_name="s",
                                  num_cores=N_CORES, num_subcores=N_SUBCORES)
    run = pl.kernel(
        _body,
        out_type=jax.ShapeDtypeStruct((n_pairs * 3,), jnp.float32),
        mesh=mesh,
        scratch_types=[
            pltpu.VMEM((N_ATOMS_WORDS,), jnp.int32),
            pltpu.VMEM((B,), jnp.int32),
            pltpu.VMEM((B,), jnp.int32),
            pltpu.VMEM((3 * B,), jnp.float32),
            pltpu.VMEM((3 * B,), jnp.float32),
        ],
        compiler_params=pltpu.CompilerParams(needs_layout_passes=False),
    )
    out = run(rw, off_flat, idx_i, idx_j)
    return out.reshape(n_pairs, 3)


# speedup vs baseline: 7.8792x; 5.9807x over previous
import jax
import jax.numpy as jnp
from jax import lax
from jax.experimental import pallas as pl
from jax.experimental.pallas import tpu as pltpu
from jax.experimental.pallas import tpu_sc as plsc

N_CORES = 2
N_SUBCORES = 16
N_WORKERS = N_CORES * N_SUBCORES

N_ATOMS = 50000
HALF = N_ATOMS // 2
N_WORDS = 3 * HALF
B = 2000

HI_MASK = -65536


def _body(rw_hbm, off_hbm, idxi_hbm, idxj_hbm, out_hbm,
          rw_v, idxi_v, idxj_v, off_v, out_v):
    wid = lax.axis_index("s") * N_CORES + lax.axis_index("c")
    n_pairs = idxi_hbm.shape[0]
    pairs_per_w = n_pairs // N_WORKERS
    nblocks = pairs_per_w // B
    w_base = wid * pairs_per_w

    pltpu.sync_copy(rw_hbm, rw_v)

    @pl.loop(0, nblocks)
    def _block(b):
        pair_base = w_base + b * B

        pltpu.sync_copy(idxi_hbm.at[pl.ds(pair_base, B)], idxi_v)
        pltpu.sync_copy(idxj_hbm.at[pl.ds(pair_base, B)], idxj_v)
        for c in range(3):
            pltpu.sync_copy(off_hbm.at[pl.ds(c * n_pairs + pair_base, B)],
                            off_v.at[pl.ds(c * B, B)])

        @pl.loop(0, B // 16)
        def _vec(t):
            base = 16 * t
            vj = idxj_v[pl.ds(base, 16)]
            vi = idxi_v[pl.ds(base, 16)]
            wj = vj >> 1
            hj = (vj & 1) == 1
            wi = vi >> 1
            hi = (vi & 1) == 1
            for c in range(3):
                aj = plsc.load_gather(rw_v, [wj + c * HALF])
                ai = plsc.load_gather(rw_v, [wi + c * HALF])
                fj = plsc.bitcast(jnp.where(hj, aj & HI_MASK, aj << 16),
                                  jnp.float32)
                fi = plsc.bitcast(jnp.where(hi, ai & HI_MASK, ai << 16),
                                  jnp.float32)
                s = pl.ds(c * B + base, 16)
                out_v[s] = fj - fi + off_v[s]

        for c in range(3):
            pltpu.sync_copy(out_v.at[pl.ds(c * B, B)],
                            out_hbm.at[pl.ds(c * n_pairs + pair_base, B)])


def kernel(R, offsets, idx_i, idx_j):
    n_pairs = idx_i.shape[0]
    rw = lax.bitcast_convert_type(
        R.astype(jnp.bfloat16).T.reshape(N_WORDS, 2), jnp.int32)
    off_flat = offsets.T.reshape(-1)
    idx_i = idx_i.astype(jnp.int32)
    idx_j = idx_j.astype(jnp.int32)

    mesh = plsc.VectorSubcoreMesh(core_axis_name="c", subcore_axis_name="s",
                                  num_cores=N_CORES, num_subcores=N_SUBCORES)
    run = pl.kernel(
        _body,
        out_type=jax.ShapeDtypeStruct((n_pairs * 3,), jnp.float32),
        mesh=mesh,
        scratch_types=[
            pltpu.VMEM((N_WORDS,), jnp.int32),
            pltpu.VMEM((B,), jnp.int32),
            pltpu.VMEM((B,), jnp.int32),
            pltpu.VMEM((3 * B,), jnp.float32),
            pltpu.VMEM((3 * B,), jnp.float32),
        ],
        compiler_params=pltpu.CompilerParams(needs_layout_passes=False),
    )
    out = run(rw, off_flat, idx_i, idx_j)
    return out.reshape(3, n_pairs).T


# speedup vs baseline: 36.5557x; 4.6395x over previous
import jax
import jax.numpy as jnp
from jax import lax
from jax.experimental import pallas as pl
from jax.experimental.pallas import tpu as pltpu
from jax.experimental.pallas import tpu_sc as plsc

N_CORES = 2
N_SUBCORES = 16
N_WORKERS = N_CORES * N_SUBCORES

N_ATOMS = 50000
HALF = N_ATOMS // 2
N_WORDS = 3 * HALF

CB = 20
BP = 128 * CB
BW = 512 * CB

HI_MASK = -65536


def _body(rw_hbm, idxi_hbm, idxj_hbm, out_hbm, rw_v, idxi_v, idxj_v, out_v):
    wid = lax.axis_index("s") * N_CORES + lax.axis_index("c")
    n_pairs = idxi_hbm.shape[0]
    nblocks = n_pairs // BP

    pltpu.sync_copy(rw_hbm, rw_v)

    @pl.loop(0, (nblocks + N_WORKERS - 1) // N_WORKERS)
    def _m(m):
        blk = wid + N_WORKERS * m

        @pl.when(blk < nblocks)
        def _():
            pair_base = BP * blk
            pltpu.sync_copy(idxi_hbm.at[pl.ds(pair_base, BP)], idxi_v)
            pltpu.sync_copy(idxj_hbm.at[pl.ds(pair_base, BP)], idxj_v)

            @pl.loop(0, BP // 16)
            def _vec(t):
                base = 16 * t
                vj = idxj_v[pl.ds(base, 16)]
                vi = idxi_v[pl.ds(base, 16)]
                wj = vj >> 1
                hj = (vj & 1) == 1
                wi = vi >> 1
                hi = (vi & 1) == 1
                u = base // 128
                lo = base % 128
                for c in range(3):
                    aj = plsc.load_gather(rw_v, [wj + c * HALF])
                    ai = plsc.load_gather(rw_v, [wi + c * HALF])
                    fj = plsc.bitcast(jnp.where(hj, aj & HI_MASK, aj << 16),
                                      jnp.float32)
                    fi = plsc.bitcast(jnp.where(hi, ai & HI_MASK, ai << 16),
                                      jnp.float32)
                    out_v[pl.ds(512 * u + 128 * c + lo, 16)] = fj - fi

            pltpu.sync_copy(out_v, out_hbm.at[pl.ds(BW * blk, BW)])


def kernel(R, offsets, idx_i, idx_j):
    n_pairs = idx_i.shape[0]
    n_chunks = n_pairs // 128
    rw = lax.bitcast_convert_type(
        R.astype(jnp.bfloat16).T.reshape(N_WORDS, 2), jnp.int32)
    idx_i = idx_i.astype(jnp.int32)
    idx_j = idx_j.astype(jnp.int32)

    mesh = plsc.VectorSubcoreMesh(core_axis_name="c", subcore_axis_name="s",
                                  num_cores=N_CORES, num_subcores=N_SUBCORES)
    run = pl.kernel(
        _body,
        out_type=jax.ShapeDtypeStruct((n_chunks * 512,), jnp.float32),
        mesh=mesh,
        scratch_types=[
            pltpu.VMEM((N_WORDS,), jnp.int32),
            pltpu.VMEM((BP,), jnp.int32),
            pltpu.VMEM((BP,), jnp.int32),
            pltpu.VMEM((BW,), jnp.float32),
        ],
        compiler_params=pltpu.CompilerParams(needs_layout_passes=False),
    )
    flat = run(rw, idx_i, idx_j)
    d = jnp.swapaxes(flat.reshape(n_chunks, 4, 128), 1, 2).reshape(n_pairs, 4)[:, :3]
    return d + offsets


# speedup vs baseline: 44.0203x; 1.2042x over previous
import jax
import jax.numpy as jnp
from jax import lax
from jax.experimental import pallas as pl
from jax.experimental.pallas import tpu as pltpu
from jax.experimental.pallas import tpu_sc as plsc

N_CORES = 2
N_SUBCORES = 16
N_WORKERS = N_CORES * N_SUBCORES

N_ATOMS = 50000
HALF = N_ATOMS // 2
N_WORDS = 3 * HALF

CB = 20
BP = 128 * CB
BW = 512 * CB

HI_MASK = -65536


def _body(rw_hbm, idxi_hbm, idxj_hbm, out_hbm, rw_v,
          ii0, ii1, jj0, jj1, ob0, ob1,
          si0, si1, sj0, sj1, so0, so1):
    wid = lax.axis_index("s") * N_CORES + lax.axis_index("c")
    n_pairs = idxi_hbm.shape[0]
    nblocks = n_pairs // BP
    nm = (nblocks + N_WORKERS - 1) // N_WORKERS

    ii = (ii0, ii1)
    jj = (jj0, jj1)
    ob = (ob0, ob1)
    si = (si0, si1)
    sj = (sj0, sj1)
    so = (so0, so1)

    def start_in(m, ph):
        blk = wid + N_WORKERS * m

        @pl.when(blk < nblocks)
        def _():
            base = BP * blk
            pltpu.async_copy(idxi_hbm.at[pl.ds(base, BP)], ii[ph], si[ph])
            pltpu.async_copy(idxj_hbm.at[pl.ds(base, BP)], jj[ph], sj[ph])

    def compute(ii_v, jj_v, ob_v):
        @pl.loop(0, BP // 16, unroll=4)
        def _vec(t):
            base = 16 * t
            vj = jj_v[pl.ds(base, 16)]
            vi = ii_v[pl.ds(base, 16)]
            wj = vj >> 1
            hj = (vj & 1) == 1
            wi = vi >> 1
            hi = (vi & 1) == 1
            u = base // 128
            lo = base % 128
            for c in range(3):
                aj = plsc.load_gather(rw_v, [wj + c * HALF])
                ai = plsc.load_gather(rw_v, [wi + c * HALF])
                fj = plsc.bitcast(jnp.where(hj, aj & HI_MASK, aj << 16),
                                  jnp.float32)
                fi = plsc.bitcast(jnp.where(hi, ai & HI_MASK, ai << 16),
                                  jnp.float32)
                ob_v[pl.ds(512 * u + 128 * c + lo, 16)] = fj - fi

    def step(m, ph):
        blk = wid + N_WORKERS * m
        start_in(m + 1, 1 - ph)

        @pl.when(blk < nblocks)
        def _():
            @pl.when(m >= 2)
            def _():
                pltpu.make_async_copy(
                    ob[ph], out_hbm.at[pl.ds(0, BW)], so[ph]).wait()
            pltpu.make_async_copy(idxi_hbm.at[pl.ds(0, BP)], ii[ph], si[ph]).wait()
            pltpu.make_async_copy(idxj_hbm.at[pl.ds(0, BP)], jj[ph], sj[ph]).wait()
            compute(ii[ph], jj[ph], ob[ph])
            pltpu.async_copy(ob[ph], out_hbm.at[pl.ds(BW * blk, BW)], so[ph])

    start_in(0, 0)
    pltpu.sync_copy(rw_hbm, rw_v)

    @pl.loop(0, nm // 2)
    def _h(h):
        step(2 * h, 0)
        step(2 * h + 1, 1)

    for ph, m in ((0, nm - 2), (1, nm - 1)):
        blk = wid + N_WORKERS * m

        @pl.when(blk < nblocks)
        def _():
            pltpu.make_async_copy(
                ob[ph], out_hbm.at[pl.ds(0, BW)], so[ph]).wait()


def kernel(R, offsets, idx_i, idx_j):
    n_pairs = idx_i.shape[0]
    n_chunks = n_pairs // 128
    rw = lax.bitcast_convert_type(
        R.astype(jnp.bfloat16).T.reshape(N_WORDS, 2), jnp.int32)
    idx_i = idx_i.astype(jnp.int32)
    idx_j = idx_j.astype(jnp.int32)

    mesh = plsc.VectorSubcoreMesh(core_axis_name="c", subcore_axis_name="s",
                                  num_cores=N_CORES, num_subcores=N_SUBCORES)
    run = pl.kernel(
        _body,
        out_type=jax.ShapeDtypeStruct((n_chunks * 512,), jnp.float32),
        mesh=mesh,
        scratch_types=[
            pltpu.VMEM((N_WORDS,), jnp.int32),
            pltpu.VMEM((BP,), jnp.int32),
            pltpu.VMEM((BP,), jnp.int32),
            pltpu.VMEM((BP,), jnp.int32),
            pltpu.VMEM((BP,), jnp.int32),
            pltpu.VMEM((BW,), jnp.float32),
            pltpu.VMEM((BW,), jnp.float32),
            pltpu.SemaphoreType.DMA,
            pltpu.SemaphoreType.DMA,
            pltpu.SemaphoreType.DMA,
            pltpu.SemaphoreType.DMA,
            pltpu.SemaphoreType.DMA,
            pltpu.SemaphoreType.DMA,
        ],
        compiler_params=pltpu.CompilerParams(needs_layout_passes=False),
    )
    flat = run(rw, idx_i, idx_j)
    d = jnp.swapaxes(flat.reshape(n_chunks, 4, 128), 1, 2).reshape(n_pairs, 4)[:, :3]
    return d + offsets


# speedup vs baseline: 82.7400x; 1.8796x over previous
import jax
import jax.numpy as jnp
from jax import lax
from jax.experimental import pallas as pl
from jax.experimental.pallas import tpu as pltpu
from jax.experimental.pallas import tpu_sc as plsc

N_CORES = 2
N_SUBCORES = 16
N_WORKERS = N_CORES * N_SUBCORES

N_ATOMS = 50000

CB = 10
BP = 128 * CB
BW = 512 * CB

HI_MASK = -65536


def _body(ta_hbm, tb_hbm, idxi_hbm, idxj_hbm, out_hbm, ta_v, tb_v,
          ii0, ii1, jj0, jj1, ob0, ob1,
          si0, si1, sj0, sj1, so0, so1):
    wid = lax.axis_index("s") * N_CORES + lax.axis_index("c")
    n_pairs = idxi_hbm.shape[0]
    nblocks = n_pairs // BP
    nm = (nblocks + N_WORKERS - 1) // N_WORKERS

    ii = (ii0, ii1)
    jj = (jj0, jj1)
    ob = (ob0, ob1)
    si = (si0, si1)
    sj = (sj0, sj1)
    so = (so0, so1)

    def start_in(m, ph):
        blk = wid + N_WORKERS * m

        @pl.when(blk < nblocks)
        def _():
            base = BP * blk
            pltpu.async_copy(idxi_hbm.at[pl.ds(base, BP)], ii[ph], si[ph])
            pltpu.async_copy(idxj_hbm.at[pl.ds(base, BP)], jj[ph], sj[ph])

    def compute(ii_v, jj_v, ob_v):
        @pl.loop(0, BP // 16, unroll=4)
        def _vec(t):
            base = 16 * t
            vj = jj_v[pl.ds(base, 16)]
            vi = ii_v[pl.ds(base, 16)]
            aj = plsc.load_gather(ta_v, [vj])
            ai = plsc.load_gather(ta_v, [vi])
            bj = plsc.load_gather(tb_v, [vj])
            bi = plsc.load_gather(tb_v, [vi])
            dx = (plsc.bitcast(aj & HI_MASK, jnp.float32)
                  - plsc.bitcast(ai & HI_MASK, jnp.float32))
            dy = (plsc.bitcast(aj << 16, jnp.float32)
                  - plsc.bitcast(ai << 16, jnp.float32))
            dz = (plsc.bitcast(bj, jnp.float32)
                  - plsc.bitcast(bi, jnp.float32))
            u = base // 128
            lo = base % 128
            ob_v[pl.ds(512 * u + lo, 16)] = dx
            ob_v[pl.ds(512 * u + 128 + lo, 16)] = dy
            ob_v[pl.ds(512 * u + 256 + lo, 16)] = dz

    def step(m, ph):
        blk = wid + N_WORKERS * m
        start_in(m + 1, 1 - ph)

        @pl.when(blk < nblocks)
        def _():
            @pl.when(m >= 2)
            def _():
                pltpu.make_async_copy(
                    ob[ph], out_hbm.at[pl.ds(0, BW)], so[ph]).wait()
            pltpu.make_async_copy(idxi_hbm.at[pl.ds(0, BP)], ii[ph], si[ph]).wait()
            pltpu.make_async_copy(idxj_hbm.at[pl.ds(0, BP)], jj[ph], sj[ph]).wait()
            compute(ii[ph], jj[ph], ob[ph])
            pltpu.async_copy(ob[ph], out_hbm.at[pl.ds(BW * blk, BW)], so[ph])

    start_in(0, 0)
    pltpu.sync_copy(ta_hbm, ta_v)
    pltpu.sync_copy(tb_hbm, tb_v)

    @pl.loop(0, nm // 2)
    def _h(h):
        step(2 * h, 0)
        step(2 * h + 1, 1)

    for ph, m in ((0, nm - 2), (1, nm - 1)):
        blk = wid + N_WORKERS * m

        @pl.when(blk < nblocks)
        def _():
            pltpu.make_async_copy(
                ob[ph], out_hbm.at[pl.ds(0, BW)], so[ph]).wait()


def kernel(R, offsets, idx_i, idx_j):
    n_pairs = idx_i.shape[0]
    n_chunks = n_pairs // 128
    u = lax.bitcast_convert_type(R.astype(jnp.bfloat16), jnp.uint16)
    x = u[:, 0].astype(jnp.uint32)
    y = u[:, 1].astype(jnp.uint32)
    z = u[:, 2].astype(jnp.uint32)
    ta = ((x << 16) | y).astype(jnp.int32)
    tb = (z << 16).astype(jnp.int32)
    idx_i = idx_i.astype(jnp.int32)
    idx_j = idx_j.astype(jnp.int32)

    mesh = plsc.VectorSubcoreMesh(core_axis_name="c", subcore_axis_name="s",
                                  num_cores=N_CORES, num_subcores=N_SUBCORES)
    run = pl.kernel(
        _body,
        out_type=jax.ShapeDtypeStruct((n_chunks * 512,), jnp.float32),
        mesh=mesh,
        scratch_types=[
            pltpu.VMEM((N_ATOMS,), jnp.int32),
            pltpu.VMEM((N_ATOMS,), jnp.int32),
            pltpu.VMEM((BP,), jnp.int32),
            pltpu.VMEM((BP,), jnp.int32),
            pltpu.VMEM((BP,), jnp.int32),
            pltpu.VMEM((BP,), jnp.int32),
            pltpu.VMEM((BW,), jnp.float32),
            pltpu.VMEM((BW,), jnp.float32),
            pltpu.SemaphoreType.DMA,
            pltpu.SemaphoreType.DMA,
            pltpu.SemaphoreType.DMA,
            pltpu.SemaphoreType.DMA,
            pltpu.SemaphoreType.DMA,
            pltpu.SemaphoreType.DMA,
        ],
        compiler_params=pltpu.CompilerParams(needs_layout_passes=False),
    )
    flat = run(ta, tb, idx_i, idx_j)
    d = jnp.swapaxes(flat.reshape(n_chunks, 4, 128), 1, 2).reshape(n_pairs, 4)[:, :3]
    return d + offsets
